# baseline (speedup 1.0000x reference)
import jax
import jax.numpy as jnp
from jax.experimental import pallas as pl


def _square_distance(a, b):
    aa = jnp.sum(a * a, -1, keepdims=True)
    bb = jnp.sum(b * b, -1, keepdims=True)
    return aa + jnp.transpose(bb, (0, 2, 1)) - 2.0 * jnp.einsum('bnc,bmc->bnm', a, b)


def _index_points(data, idx):
    return jax.vmap(lambda d, i: d[i])(data, idx)


def _bn(x, gamma, beta, axes):
    m = jnp.mean(x, axes, keepdims=True)
    v = jnp.mean((x - m) ** 2, axes, keepdims=True)
    return (x - m) / jnp.sqrt(v + 1e-5) * gamma + beta


def _knn_index(xyz, k):
    dist = _square_distance(xyz, xyz)
    dist = jnp.where(dist < 1e-8, jnp.inf, dist)
    _, idx = jax.lax.top_k(-dist, k)
    B, N, _ = xyz.shape
    self_idx = jnp.broadcast_to(jnp.arange(N, dtype=idx.dtype)[None, :, None], (B, N, 1))
    return jnp.concatenate([self_idx, idx], -1)


def _interp_conv(xyz, data, w1, b1, g, be, w2, b2, cw, cb, ks, ls, depthwise):
    B, N, dim = data.shape
    li = _knn_index(xyz, ls - 1)
    local_data = _index_points(data, li)
    local_xyz = _index_points(xyz, li)
    rel = (local_xyz - xyz[:, :, None, :]).reshape(B * N * ls, 3)
    h = rel @ w1.T + b1
    h = jax.nn.relu(_bn(h, g, be, (0,)))
    alpha = jax.nn.softmax(h @ w2.T + b2, -1).reshape(B * N, ls, ks)
    ld = jnp.transpose(local_data.reshape(B * N, ls, dim), (0, 2, 1))
    m = jnp.einsum('bcl,blk->bck', ld, alpha)
    if depthwise:
        out = jnp.einsum('bck,ck->bc', m, cw[:, 0, :]) + cb
    else:
        out = jnp.einsum('bk,ok->bo', m[:, 0, :], cw[:, 0, :]) + cb
    return out.reshape(B, N, -1)


def _lbr(xyz, data, w, g, be):
    x = jnp.concatenate([data, xyz], -1) @ w.T
    return jax.nn.relu(_bn(x, g, be, (0, 1)))


def _maxpool(xyz, data, ls):
    li = _knn_index(xyz, ls)
    return jnp.max(_index_points(data, li), 2)


def _fc_kernel(x_ref, w_ref, b_ref, o_ref):
    o_ref[...] = x_ref[...] @ w_ref[...].T + b_ref[...][None, :]


def kernel(points, params):
    p = params
    xyz = jnp.transpose(points, (0, 2, 1))
    B, N, _ = xyz.shape
    data = jnp.ones((B, N, 1), points.dtype)
    data = _interp_conv(xyz, data, p['ic1_w1'], p['ic1_b1'], p['ic1_g'], p['ic1_be'], p['ic1_w2'], p['ic1_b2'], p['ic1_cw'], p['ic1_cb'], 32, 32, False)
    data = _lbr(xyz, data, p['lbr1_w'], p['lbr1_g'], p['lbr1_be'])
    data = _maxpool(xyz, data, 32)
    xyz, data = xyz[:, :N // 4], data[:, :N // 4]
    data = _lbr(xyz, data, p['lbr2_w'], p['lbr2_g'], p['lbr2_be'])
    data = _interp_conv(xyz, data, p['ic2_w1'], p['ic2_b1'], p['ic2_g'], p['ic2_be'], p['ic2_w2'], p['ic2_b2'], p['ic2_cw'], p['ic2_cb'], 16, 16, True)
    data = _lbr(xyz, data, p['lbr3_w'], p['lbr3_g'], p['lbr3_be'])
    data = _maxpool(xyz, data, 16)
    xyz, data = xyz[:, :N // 16], data[:, :N // 16]
    data = _lbr(xyz, data, p['lbr4_w'], p['lbr4_g'], p['lbr4_be'])
    data = _interp_conv(xyz, data, p['ic3_w1'], p['ic3_b1'], p['ic3_g'], p['ic3_be'], p['ic3_w2'], p['ic3_b2'], p['ic3_cw'], p['ic3_cb'], 16, 16, True)
    data = _lbr(xyz, data, p['lbr5_w'], p['lbr5_g'], p['lbr5_be'])
    x = jnp.mean(data, 1)
    return pl.pallas_call(
        _fc_kernel,
        out_shape=jax.ShapeDtypeStruct((x.shape[0], p['fc_w'].shape[0]), x.dtype),
    )(x, p['fc_w'], p['fc_b'])


# speedup vs baseline: 1.1750x; 1.1750x over previous
import functools

import jax
import jax.numpy as jnp
from jax.experimental import pallas as pl


def _knn_body(xyz_ref, idx_ref, *, n, k):
    x = xyz_ref[0]
    aa = jnp.sum(x * x, axis=-1, keepdims=True)
    dist = aa + aa.reshape(1, n) - 2.0 * jax.lax.dot_general(
        x, x, (((1,), (1,)), ((), ())), preferred_element_type=jnp.float32)
    inf = jnp.float32(jnp.inf)
    dist = jnp.where(dist < 1e-8, inf, dist)
    col = jax.lax.broadcasted_iota(jnp.int32, (n, n), 1)
    row = jax.lax.broadcasted_iota(jnp.int32, (n, 1), 0)
    cols = [row]
    for _ in range(k):
        m = jnp.min(dist, axis=1, keepdims=True)
        sel = jnp.where(dist == m, col, n)
        j = jnp.min(sel, axis=1, keepdims=True)
        cols.append(jnp.minimum(j, n - 1))
        dist = jnp.where(col == j, inf, dist)
    idx_ref[0] = jnp.concatenate(cols, axis=1)


def _knn_index(xyz, k):
    B, N, _ = xyz.shape
    return pl.pallas_call(
        functools.partial(_knn_body, n=N, k=k),
        grid=(B,),
        in_specs=[pl.BlockSpec((1, N, 3), lambda b: (b, 0, 0))],
        out_specs=pl.BlockSpec((1, N, k + 1), lambda b: (b, 0, 0)),
        out_shape=jax.ShapeDtypeStruct((B, N, k + 1), jnp.int32),
    )(xyz)


def _index_points(data, idx):
    return jax.vmap(lambda d, i: d[i])(data, idx)


def _bn(x, gamma, beta, axes):
    m = jnp.mean(x, axes, keepdims=True)
    v = jnp.mean((x - m) ** 2, axes, keepdims=True)
    return (x - m) / jnp.sqrt(v + 1e-5) * gamma + beta


def _interp_conv(xyz, data, li, w1, b1, g, be, w2, b2, cw, cb, ks, ls, depthwise):
    B, N, dim = data.shape
    local_data = _index_points(data, li)
    local_xyz = _index_points(xyz, li)
    rel = (local_xyz - xyz[:, :, None, :]).reshape(B * N * ls, 3)
    h = rel @ w1.T + b1
    h = jax.nn.relu(_bn(h, g, be, (0,)))
    alpha = jax.nn.softmax(h @ w2.T + b2, -1).reshape(B * N, ls, ks)
    ld = jnp.transpose(local_data.reshape(B * N, ls, dim), (0, 2, 1))
    m = jnp.einsum('bcl,blk->bck', ld, alpha)
    if depthwise:
        out = jnp.einsum('bck,ck->bc', m, cw[:, 0, :]) + cb
    else:
        out = jnp.einsum('bk,ok->bo', m[:, 0, :], cw[:, 0, :]) + cb
    return out.reshape(B, N, -1)


def _lbr(xyz, data, w, g, be):
    x = jnp.concatenate([data, xyz], -1) @ w.T
    return jax.nn.relu(_bn(x, g, be, (0, 1)))


def _fc_kernel(x_ref, w_ref, b_ref, o_ref):
    o_ref[...] = x_ref[...] @ w_ref[...].T + b_ref[...][None, :]


def kernel(points, params):
    p = params
    xyz = jnp.transpose(points, (0, 2, 1))
    B, N, _ = xyz.shape
    data = jnp.ones((B, N, 1), points.dtype)

    idx32 = _knn_index(xyz, 32)
    data = _interp_conv(xyz, data, idx32[:, :, :32],
                        p['ic1_w1'], p['ic1_b1'], p['ic1_g'], p['ic1_be'],
                        p['ic1_w2'], p['ic1_b2'], p['ic1_cw'], p['ic1_cb'],
                        32, 32, False)
    data = _lbr(xyz, data, p['lbr1_w'], p['lbr1_g'], p['lbr1_be'])
    data = jnp.max(_index_points(data, idx32), 2)
    xyz, data = xyz[:, :N // 4], data[:, :N // 4]

    data = _lbr(xyz, data, p['lbr2_w'], p['lbr2_g'], p['lbr2_be'])
    idx16 = _knn_index(xyz, 16)
    data = _interp_conv(xyz, data, idx16[:, :, :16],
                        p['ic2_w1'], p['ic2_b1'], p['ic2_g'], p['ic2_be'],
                        p['ic2_w2'], p['ic2_b2'], p['ic2_cw'], p['ic2_cb'],
                        16, 16, True)
    data = _lbr(xyz, data, p['lbr3_w'], p['lbr3_g'], p['lbr3_be'])
    data = jnp.max(_index_points(data, idx16), 2)
    xyz, data = xyz[:, :N // 16], data[:, :N // 16]

    data = _lbr(xyz, data, p['lbr4_w'], p['lbr4_g'], p['lbr4_be'])
    idx15 = _knn_index(xyz, 15)
    data = _interp_conv(xyz, data, idx15,
                        p['ic3_w1'], p['ic3_b1'], p['ic3_g'], p['ic3_be'],
                        p['ic3_w2'], p['ic3_b2'], p['ic3_cw'], p['ic3_cb'],
                        16, 16, True)
    data = _lbr(xyz, data, p['lbr5_w'], p['lbr5_g'], p['lbr5_be'])

    x = jnp.mean(data, 1)
    return pl.pallas_call(
        _fc_kernel,
        out_shape=jax.ShapeDtypeStruct((x.shape[0], p['fc_w'].shape[0]), x.dtype),
    )(x, p['fc_w'], p['fc_b'])
